# baseline (speedup 1.0000x reference)
import functools
import jax
import jax.numpy as jnp
from jax import lax
from jax.experimental import pallas as pl
from jax.experimental.pallas import tpu as pltpu
from jax.experimental.pallas import tpu_sc as plsc

N = 10000
E = 320000
D_IN = 128
HID = 128

NC = 2
NS = 16
NW = NC * NS
K = 64
NPAD = 10240
DDST = NPAD - 8
CPT_L1 = 320
EPAD = NS * CPT_L1 * K
NCHUNKS = EPAD // K
CPT_L2 = NCHUNKS // NW
NQ = 8
NBR = 4


@functools.cache
def _mesh():
  return plsc.VectorSubcoreMesh(
      core_axis_name="c", subcore_axis_name="s", num_cores=NC, num_subcores=NS
  )


def _zero_vmem2d(buf, rows, cols):
  zv = jnp.zeros((16,), jnp.float32)

  @pl.loop(0, rows)
  def _(r):
    for k in range(cols // 16):
      buf[r, pl.ds(k * 16, 16)] = zv


def _zero_and_barrier(zslab, z_sh, s):
  _zero_vmem2d(zslab, 8, HID)
  rows_per_tile = NPAD // NS
  for t in range(rows_per_tile // 8):
    pltpu.sync_copy(zslab, z_sh.at[pl.ds(s * rows_per_tile + t * 8, 8)])
  plsc.subcore_barrier()


def _copy_out(z_sh, z_hbm, c, s):
  rows_out = NPAD // NS
  pltpu.sync_copy(
      z_sh.at[pl.ds(s * rows_out, rows_out)],
      z_hbm.at[c, pl.ds(s * rows_out, rows_out)],
  )


DEGW = 128


@functools.cache
def _deg_kernel():
  return pl.kernel(
      _deg_body,
      out_type=jax.ShapeDtypeStruct((NC, NPAD, DEGW), jnp.float32),
      mesh=_mesh(),
      scratch_types=[
          pltpu.VMEM((K,), jnp.int32),
          pltpu.VMEM((K,), jnp.int32),
          pltpu.VMEM((K, DEGW), jnp.float32),
          pltpu.VMEM((8, DEGW), jnp.float32),
          pltpu.VMEM_SHARED((NPAD, DEGW), jnp.float32),
          pltpu.SemaphoreType.DMA,
          pltpu.SemaphoreType.DMA,
      ],
  )


def _deg_body(dst2_hbm, deg_hbm, didx0, didx1, ones, zslab, deg_sh, dm0, dm1):
  c = lax.axis_index("c")
  s = lax.axis_index("s")
  didx = [didx0, didx1]
  dms = [dm0, dm1]

  ov = jnp.full((16,), 1.0, jnp.float32)

  @pl.loop(0, K)
  def _(r):
    for k in range(DEGW // 16):
      ones[r, pl.ds(k * 16, 16)] = ov

  _zero_and_barrier(zslab, deg_sh, s)

  wid = c * NS + s
  cbase = wid * CPT_L2

  pltpu.async_copy(dst2_hbm.at[cbase], didx[0], dms[0])

  @pl.loop(0, CPT_L2 // 2)
  def _(g):
    j0 = 2 * g
    for p in range(2):
      j = j0 + p
      pltpu.make_async_copy(dst2_hbm.at[0], didx[p], dms[p]).wait()

      @pl.when(j + 1 < CPT_L2)
      def _():
        pltpu.async_copy(dst2_hbm.at[cbase + j + 1], didx[1 - p], dms[1 - p])

      pltpu.sync_copy(ones, deg_sh.at[didx[p]], add=True)

  plsc.subcore_barrier()
  _copy_out(deg_sh, deg_hbm, c, s)


def _edge_pass(y_ref, src2_hbm, dst2_hbm, cbase, n, sidx, didx, isems, idsems,
               rows, gsems, z_sh):
  def idx_start(j, q):
    pltpu.async_copy(src2_hbm.at[cbase + j], sidx[q], isems[q])
    pltpu.async_copy(dst2_hbm.at[cbase + j], didx[q], idsems[q])

  def idx_wait(q):
    pltpu.make_async_copy(src2_hbm.at[0], sidx[q], isems[q]).wait()
    pltpu.make_async_copy(dst2_hbm.at[0], didx[q], idsems[q]).wait()

  def gather_start(q, b):
    pltpu.async_copy(y_ref.at[sidx[q]], rows[b], gsems[b])

  def gather_wait(b):
    pltpu.make_async_copy(y_ref.at[sidx[0]], rows[b], gsems[b]).wait()

  def scatter(b, q):
    pltpu.sync_copy(rows[b], z_sh.at[didx[q]], add=True)

  for q in range(NQ):
    idx_start(q, q)
  for j in range(NBR):
    idx_wait(j)
    gather_start(j, j)

  @pl.loop(0, (n - NQ) // NQ)
  def _(g):
    j0 = g * NQ
    for p in range(NQ):
      j = j0 + p
      b = p % NBR
      gather_wait(b)
      scatter(b, p)
      idx_start(j + NQ, p)
      qq = (p + NBR) % NQ
      idx_wait(qq)
      gather_start(qq, b)

  for p in range(NQ):
    b = p % NBR
    gather_wait(b)
    scatter(b, p)
    if p < NQ - NBR:
      qq = (p + NBR) % NQ
      idx_wait(qq)
      gather_start(qq, b)


def _scatter_scratch():
  return (
      [pltpu.VMEM((K,), jnp.int32) for _ in range(2 * NQ)]
      + [pltpu.VMEM((K, HID), jnp.float32) for _ in range(NBR)]
      + [pltpu.VMEM((8, HID), jnp.float32),
         pltpu.VMEM_SHARED((NPAD, HID), jnp.float32)]
      + [pltpu.SemaphoreType.DMA for _ in range(2 * NQ + NBR)]
  )


@functools.cache
def _scatter_l1():
  return pl.kernel(
      _scatter_l1_body,
      out_type=jax.ShapeDtypeStruct((NC, NPAD, HID), jnp.float32),
      mesh=_mesh(),
      scratch_types=_scatter_scratch(),
  )


def _scatter_l1_body(ya_hbm, yb_hbm, src2_hbm, dst2_hbm, z_hbm, *scr):
  c = lax.axis_index("c")
  s = lax.axis_index("s")
  sidx, didx = list(scr[:NQ]), list(scr[NQ:2 * NQ])
  rows = list(scr[2 * NQ:2 * NQ + NBR])
  zslab, z_sh = scr[2 * NQ + NBR], scr[2 * NQ + NBR + 1]
  sems = scr[2 * NQ + NBR + 2:]
  isems, idsems = list(sems[:NQ]), list(sems[NQ:2 * NQ])
  gsems = list(sems[2 * NQ:])
  _zero_and_barrier(zslab, z_sh, s)

  cbase = s * CPT_L1

  @pl.when(c == 0)
  def _():
    _edge_pass(ya_hbm, src2_hbm, dst2_hbm, cbase, CPT_L1, sidx, didx, isems,
               idsems, rows, gsems, z_sh)

  @pl.when(c == 1)
  def _():
    _edge_pass(yb_hbm, src2_hbm, dst2_hbm, cbase, CPT_L1, sidx, didx, isems,
               idsems, rows, gsems, z_sh)

  plsc.subcore_barrier()
  _copy_out(z_sh, z_hbm, c, s)


@functools.cache
def _scatter_l2():
  return pl.kernel(
      _scatter_l2_body,
      out_type=jax.ShapeDtypeStruct((NC, NPAD, HID), jnp.float32),
      mesh=_mesh(),
      scratch_types=_scatter_scratch(),
  )


def _scatter_l2_body(y_hbm, src2_hbm, dst2_hbm, z_hbm, *scr):
  c = lax.axis_index("c")
  s = lax.axis_index("s")
  sidx, didx = list(scr[:NQ]), list(scr[NQ:2 * NQ])
  rows = list(scr[2 * NQ:2 * NQ + NBR])
  zslab, z_sh = scr[2 * NQ + NBR], scr[2 * NQ + NBR + 1]
  sems = scr[2 * NQ + NBR + 2:]
  isems, idsems = list(sems[:NQ]), list(sems[NQ:2 * NQ])
  gsems = list(sems[2 * NQ:])
  _zero_and_barrier(zslab, z_sh, s)

  wid = c * NS + s
  cbase = wid * CPT_L2
  _edge_pass(y_hbm, src2_hbm, dst2_hbm, cbase, CPT_L2, sidx, didx, isems,
             idsems, rows, gsems, z_sh)

  plsc.subcore_barrier()
  _copy_out(z_sh, z_hbm, c, s)


RB = 1000
GRID = N // RB


def _tc12_body(x_ref, w_ref, dega_ref, degb_ref,
               ya_ref, yb_ref, dinv_ref, hn_ref, hs_ref, hg_ref):
  i = pl.program_id(0)
  xw = jnp.dot(x_ref[...], w_ref[...], preferred_element_type=jnp.float32)
  deg = dega_ref[...] + degb_ref[...] + 1.0
  dinv = lax.rsqrt(deg)
  dinv_ref[...] = dinv
  y = xw[:, : 2 * HID] * dinv
  ya_ref[...] = y[:, :HID]
  yb_ref[...] = y[:, HID:]
  hn = xw[:, 2 * HID : 3 * HID]
  hn_ref[...] = hn
  hs_ref[...] = xw[:, 3 * HID :]

  @pl.when(i == 0)
  def _():
    hg_ref[...] = jnp.zeros_like(hg_ref)

  hg_ref[...] += jnp.sum(hn, axis=0, keepdims=True) * (1.0 / N)


def _tc3_body(za_ref, zb_ref, ya_ref, yb_ref, dinv_ref, b1_ref, w2_ref, y2_ref):
  dinv = dinv_ref[...]
  h = jnp.concatenate(
      [za_ref[0] + ya_ref[...], zb_ref[0] + yb_ref[...]], axis=1
  ) * dinv + b1_ref[...]
  h = jnp.maximum(h, 0.0)
  y2_ref[...] = jnp.dot(h, w2_ref[...], preferred_element_type=jnp.float32) * dinv


def _tc4_body(za_ref, zb_ref, y2_ref, dinv_ref, b2_ref, out_ref):
  v = (za_ref[0] + zb_ref[0] + y2_ref[...]) * dinv_ref[...] + b2_ref[...]
  out_ref[...] = jnp.maximum(v, 0.0)


def _row_spec(width):
  return pl.BlockSpec((RB, width), lambda i: (i, 0))


def _part_spec(width):
  def mk(p):
    return pl.BlockSpec((1, RB, width), lambda i, p=p: (p, i, 0))
  return mk


def kernel(x, edge_index, W1, b1, W2, b2, Wn, Ws):
  wcat = jnp.concatenate([W1, Wn, Ws], axis=1)
  npad_e = EPAD - E
  fill = jnp.arange(npad_e, dtype=jnp.int32)
  src2 = jnp.concatenate(
      [edge_index[0], fill % N]).reshape(NCHUNKS, K)
  dst2 = jnp.concatenate(
      [edge_index[1], N + fill % (NPAD - N - 8)]).reshape(NCHUNKS, K)

  degp = _deg_kernel()(dst2)
  dega = degp[0, :N, 0:1]
  degb = degp[1, :N, 0:1]

  col_spec = pl.BlockSpec((RB, 1), lambda i: (i, 0))
  y1a, y1b, dinv, h_node, h_sub, h_graph = pl.pallas_call(
      _tc12_body,
      grid=(GRID,),
      in_specs=[
          _row_spec(D_IN),
          pl.BlockSpec((D_IN, 4 * HID), lambda i: (0, 0)),
          col_spec,
          col_spec,
      ],
      out_specs=[
          _row_spec(HID),
          _row_spec(HID),
          col_spec,
          _row_spec(HID),
          _row_spec(HID),
          pl.BlockSpec((1, HID), lambda i: (0, 0)),
      ],
      out_shape=[
          jax.ShapeDtypeStruct((N, HID), jnp.float32),
          jax.ShapeDtypeStruct((N, HID), jnp.float32),
          jax.ShapeDtypeStruct((N, 1), jnp.float32),
          jax.ShapeDtypeStruct((N, HID), jnp.float32),
          jax.ShapeDtypeStruct((N, HID), jnp.float32),
          jax.ShapeDtypeStruct((1, HID), jnp.float32),
      ],
      compiler_params=pltpu.CompilerParams(
          dimension_semantics=("arbitrary",)
      ),
  )(x, wcat, dega, degb)

  z1 = _scatter_l1()(y1a, y1b, src2, dst2)

  zspec = _part_spec(HID)
  y2 = pl.pallas_call(
      _tc3_body,
      grid=(GRID,),
      in_specs=[
          zspec(0),
          zspec(1),
          _row_spec(HID),
          _row_spec(HID),
          col_spec,
          pl.BlockSpec((1, 2 * HID), lambda i: (0, 0)),
          pl.BlockSpec((2 * HID, HID), lambda i: (0, 0)),
      ],
      out_specs=_row_spec(HID),
      out_shape=jax.ShapeDtypeStruct((N, HID), jnp.float32),
  )(z1, z1, y1a, y1b, dinv, b1.reshape(1, 2 * HID), W2)

  z2 = _scatter_l2()(y2, src2, dst2)

  h_gnn = pl.pallas_call(
      _tc4_body,
      grid=(GRID,),
      in_specs=[
          zspec(0),
          zspec(1),
          _row_spec(HID),
          col_spec,
          pl.BlockSpec((1, HID), lambda i: (0, 0)),
      ],
      out_specs=_row_spec(HID),
      out_shape=jax.ShapeDtypeStruct((N, HID), jnp.float32),
  )(z2, z2, y2, dinv, b2.reshape(1, HID))

  return (h_gnn, h_node, h_sub, h_graph)


# speedup vs baseline: 1.0492x; 1.0492x over previous
import functools
import jax
import jax.numpy as jnp
from jax import lax
from jax.experimental import pallas as pl
from jax.experimental.pallas import tpu as pltpu
from jax.experimental.pallas import tpu_sc as plsc

N = 10000
E = 320000
D_IN = 128
HID = 128

NC = 2
NS = 16
NW = NC * NS
K = 64
NPAD = 10240
DDST = NPAD - 8
CPT_L1 = 320
EPAD = NS * CPT_L1 * K
NCHUNKS = EPAD // K
CPT_L2 = NCHUNKS // NW
NQ = 8
NBR = 4


@functools.cache
def _mesh():
  return plsc.VectorSubcoreMesh(
      core_axis_name="c", subcore_axis_name="s", num_cores=NC, num_subcores=NS
  )


def _zero_vmem2d(buf, rows, cols):
  zv = jnp.zeros((16,), jnp.float32)

  @pl.loop(0, rows)
  def _(r):
    for k in range(cols // 16):
      buf[r, pl.ds(k * 16, 16)] = zv


def _zero_and_barrier(zslab, z_sh, s):
  _zero_vmem2d(zslab, 8, HID)
  rows_per_tile = NPAD // NS
  for t in range(rows_per_tile // 8):
    pltpu.sync_copy(zslab, z_sh.at[pl.ds(s * rows_per_tile + t * 8, 8)])
  plsc.subcore_barrier()


def _copy_out(z_sh, z_hbm, c, s):
  rows_out = NPAD // NS
  pltpu.sync_copy(
      z_sh.at[pl.ds(s * rows_out, rows_out)],
      z_hbm.at[c, pl.ds(s * rows_out, rows_out)],
  )


DEGW = 128
KD = 128
CPT_DEG = EPAD // KD // NW


@functools.cache
def _deg_kernel():
  return pl.kernel(
      _deg_body,
      out_type=jax.ShapeDtypeStruct((NC, NPAD, DEGW), jnp.float32),
      mesh=_mesh(),
      scratch_types=[
          pltpu.VMEM((KD,), jnp.int32),
          pltpu.VMEM((KD,), jnp.int32),
          pltpu.VMEM((KD, DEGW), jnp.float32),
          pltpu.VMEM((8, DEGW), jnp.float32),
          pltpu.VMEM_SHARED((NPAD, DEGW), jnp.float32),
          pltpu.SemaphoreType.DMA,
          pltpu.SemaphoreType.DMA,
      ],
  )


def _deg_body(dst2_hbm, deg_hbm, didx0, didx1, ones, zslab, deg_sh, dm0, dm1):
  c = lax.axis_index("c")
  s = lax.axis_index("s")
  didx = [didx0, didx1]
  dms = [dm0, dm1]

  ov = jnp.full((16,), 1.0, jnp.float32)

  @pl.loop(0, KD)
  def _(r):
    for k in range(DEGW // 16):
      ones[r, pl.ds(k * 16, 16)] = ov

  _zero_and_barrier(zslab, deg_sh, s)

  wid = c * NS + s
  cbase = wid * CPT_DEG

  pltpu.async_copy(dst2_hbm.at[cbase], didx[0], dms[0])

  @pl.loop(0, CPT_DEG // 2)
  def _(g):
    j0 = 2 * g
    for p in range(2):
      j = j0 + p
      pltpu.make_async_copy(dst2_hbm.at[0], didx[p], dms[p]).wait()

      @pl.when(j + 1 < CPT_DEG)
      def _():
        pltpu.async_copy(dst2_hbm.at[cbase + j + 1], didx[1 - p], dms[1 - p])

      pltpu.sync_copy(ones, deg_sh.at[didx[p]], add=True)

  plsc.subcore_barrier()
  _copy_out(deg_sh, deg_hbm, c, s)


def _edge_pass(y_ref, src2_hbm, dst2_hbm, cbase, n, sidx, didx, isems, idsems,
               rows, gsems, z_sh):
  def idx_start(j, q):
    pltpu.async_copy(src2_hbm.at[cbase + j], sidx[q], isems[q])
    pltpu.async_copy(dst2_hbm.at[cbase + j], didx[q], idsems[q])

  def idx_wait(q):
    pltpu.make_async_copy(src2_hbm.at[0], sidx[q], isems[q]).wait()
    pltpu.make_async_copy(dst2_hbm.at[0], didx[q], idsems[q]).wait()

  def gather_start(q, b):
    pltpu.async_copy(y_ref.at[sidx[q]], rows[b], gsems[b])

  def gather_wait(b):
    pltpu.make_async_copy(y_ref.at[sidx[0]], rows[b], gsems[b]).wait()

  def scatter(b, q):
    pltpu.sync_copy(rows[b], z_sh.at[didx[q]], add=True)

  for q in range(NQ):
    idx_start(q, q)
  for j in range(NBR):
    idx_wait(j)
    gather_start(j, j)

  @pl.loop(0, (n - NQ) // NQ)
  def _(g):
    j0 = g * NQ
    for p in range(NQ):
      j = j0 + p
      b = p % NBR
      gather_wait(b)
      scatter(b, p)
      idx_start(j + NQ, p)
      qq = (p + NBR) % NQ
      idx_wait(qq)
      gather_start(qq, b)

  for p in range(NQ):
    b = p % NBR
    gather_wait(b)
    scatter(b, p)
    if p < NQ - NBR:
      qq = (p + NBR) % NQ
      idx_wait(qq)
      gather_start(qq, b)


def _scatter_scratch():
  return (
      [pltpu.VMEM((K,), jnp.int32) for _ in range(2 * NQ)]
      + [pltpu.VMEM((K, HID), jnp.float32) for _ in range(NBR)]
      + [pltpu.VMEM((8, HID), jnp.float32),
         pltpu.VMEM_SHARED((NPAD, HID), jnp.float32)]
      + [pltpu.SemaphoreType.DMA for _ in range(2 * NQ + NBR)]
  )


@functools.cache
def _scatter_l1():
  return pl.kernel(
      _scatter_l1_body,
      out_type=jax.ShapeDtypeStruct((NC, NPAD, HID), jnp.float32),
      mesh=_mesh(),
      scratch_types=_scatter_scratch(),
  )


def _scatter_l1_body(ya_hbm, yb_hbm, src2_hbm, dst2_hbm, z_hbm, *scr):
  c = lax.axis_index("c")
  s = lax.axis_index("s")
  sidx, didx = list(scr[:NQ]), list(scr[NQ:2 * NQ])
  rows = list(scr[2 * NQ:2 * NQ + NBR])
  zslab, z_sh = scr[2 * NQ + NBR], scr[2 * NQ + NBR + 1]
  sems = scr[2 * NQ + NBR + 2:]
  isems, idsems = list(sems[:NQ]), list(sems[NQ:2 * NQ])
  gsems = list(sems[2 * NQ:])
  _zero_and_barrier(zslab, z_sh, s)

  cbase = s * CPT_L1

  @pl.when(c == 0)
  def _():
    _edge_pass(ya_hbm, src2_hbm, dst2_hbm, cbase, CPT_L1, sidx, didx, isems,
               idsems, rows, gsems, z_sh)

  @pl.when(c == 1)
  def _():
    _edge_pass(yb_hbm, src2_hbm, dst2_hbm, cbase, CPT_L1, sidx, didx, isems,
               idsems, rows, gsems, z_sh)

  plsc.subcore_barrier()
  _copy_out(z_sh, z_hbm, c, s)


@functools.cache
def _scatter_l2():
  return pl.kernel(
      _scatter_l2_body,
      out_type=jax.ShapeDtypeStruct((NC, NPAD, HID), jnp.float32),
      mesh=_mesh(),
      scratch_types=_scatter_scratch(),
  )


def _scatter_l2_body(y_hbm, src2_hbm, dst2_hbm, z_hbm, *scr):
  c = lax.axis_index("c")
  s = lax.axis_index("s")
  sidx, didx = list(scr[:NQ]), list(scr[NQ:2 * NQ])
  rows = list(scr[2 * NQ:2 * NQ + NBR])
  zslab, z_sh = scr[2 * NQ + NBR], scr[2 * NQ + NBR + 1]
  sems = scr[2 * NQ + NBR + 2:]
  isems, idsems = list(sems[:NQ]), list(sems[NQ:2 * NQ])
  gsems = list(sems[2 * NQ:])
  _zero_and_barrier(zslab, z_sh, s)

  wid = c * NS + s
  cbase = wid * CPT_L2
  _edge_pass(y_hbm, src2_hbm, dst2_hbm, cbase, CPT_L2, sidx, didx, isems,
             idsems, rows, gsems, z_sh)

  plsc.subcore_barrier()
  _copy_out(z_sh, z_hbm, c, s)


RB = 1000
GRID = N // RB


def _tc12_body(x_ref, w_ref, dega_ref, degb_ref,
               ya_ref, yb_ref, dinv_ref, hn_ref, hs_ref, hg_ref):
  i = pl.program_id(0)
  xw = jnp.dot(x_ref[...], w_ref[...], preferred_element_type=jnp.float32)
  deg = dega_ref[...] + degb_ref[...] + 1.0
  dinv = lax.rsqrt(deg)
  dinv_ref[...] = dinv
  y = xw[:, : 2 * HID] * dinv
  ya_ref[...] = y[:, :HID]
  yb_ref[...] = y[:, HID:]
  hn = xw[:, 2 * HID : 3 * HID]
  hn_ref[...] = hn
  hs_ref[...] = xw[:, 3 * HID :]

  @pl.when(i == 0)
  def _():
    hg_ref[...] = jnp.zeros_like(hg_ref)

  hg_ref[...] += jnp.sum(hn, axis=0, keepdims=True) * (1.0 / N)


def _tc3_body(za_ref, zb_ref, ya_ref, yb_ref, dinv_ref, b1_ref, w2_ref, y2_ref):
  dinv = dinv_ref[...]
  h = jnp.concatenate(
      [za_ref[0] + ya_ref[...], zb_ref[0] + yb_ref[...]], axis=1
  ) * dinv + b1_ref[...]
  h = jnp.maximum(h, 0.0)
  y2_ref[...] = jnp.dot(h, w2_ref[...], preferred_element_type=jnp.float32) * dinv


def _tc4_body(za_ref, zb_ref, y2_ref, dinv_ref, b2_ref, out_ref):
  v = (za_ref[0] + zb_ref[0] + y2_ref[...]) * dinv_ref[...] + b2_ref[...]
  out_ref[...] = jnp.maximum(v, 0.0)


def _row_spec(width):
  return pl.BlockSpec((RB, width), lambda i: (i, 0))


def _part_spec(width):
  def mk(p):
    return pl.BlockSpec((1, RB, width), lambda i, p=p: (p, i, 0))
  return mk


def kernel(x, edge_index, W1, b1, W2, b2, Wn, Ws):
  wcat = jnp.concatenate([W1, Wn, Ws], axis=1)
  npad_e = EPAD - E
  fill = jnp.arange(npad_e, dtype=jnp.int32)
  src2 = jnp.concatenate(
      [edge_index[0], fill % N]).reshape(NCHUNKS, K)
  dst2 = jnp.concatenate(
      [edge_index[1], N + fill % (NPAD - N - 8)]).reshape(NCHUNKS, K)

  degp = _deg_kernel()(dst2.reshape(EPAD // KD, KD))
  dega = degp[0, :N, 0:1]
  degb = degp[1, :N, 0:1]

  col_spec = pl.BlockSpec((RB, 1), lambda i: (i, 0))
  y1a, y1b, dinv, h_node, h_sub, h_graph = pl.pallas_call(
      _tc12_body,
      grid=(GRID,),
      in_specs=[
          _row_spec(D_IN),
          pl.BlockSpec((D_IN, 4 * HID), lambda i: (0, 0)),
          col_spec,
          col_spec,
      ],
      out_specs=[
          _row_spec(HID),
          _row_spec(HID),
          col_spec,
          _row_spec(HID),
          _row_spec(HID),
          pl.BlockSpec((1, HID), lambda i: (0, 0)),
      ],
      out_shape=[
          jax.ShapeDtypeStruct((N, HID), jnp.float32),
          jax.ShapeDtypeStruct((N, HID), jnp.float32),
          jax.ShapeDtypeStruct((N, 1), jnp.float32),
          jax.ShapeDtypeStruct((N, HID), jnp.float32),
          jax.ShapeDtypeStruct((N, HID), jnp.float32),
          jax.ShapeDtypeStruct((1, HID), jnp.float32),
      ],
      compiler_params=pltpu.CompilerParams(
          dimension_semantics=("arbitrary",)
      ),
  )(x, wcat, dega, degb)

  z1 = _scatter_l1()(y1a, y1b, src2, dst2)

  zspec = _part_spec(HID)
  y2 = pl.pallas_call(
      _tc3_body,
      grid=(GRID,),
      in_specs=[
          zspec(0),
          zspec(1),
          _row_spec(HID),
          _row_spec(HID),
          col_spec,
          pl.BlockSpec((1, 2 * HID), lambda i: (0, 0)),
          pl.BlockSpec((2 * HID, HID), lambda i: (0, 0)),
      ],
      out_specs=_row_spec(HID),
      out_shape=jax.ShapeDtypeStruct((N, HID), jnp.float32),
  )(z1, z1, y1a, y1b, dinv, b1.reshape(1, 2 * HID), W2)

  z2 = _scatter_l2()(y2, src2, dst2)

  h_gnn = pl.pallas_call(
      _tc4_body,
      grid=(GRID,),
      in_specs=[
          zspec(0),
          zspec(1),
          _row_spec(HID),
          col_spec,
          pl.BlockSpec((1, HID), lambda i: (0, 0)),
      ],
      out_specs=_row_spec(HID),
      out_shape=jax.ShapeDtypeStruct((N, HID), jnp.float32),
  )(z2, z2, y2, dinv, b2.reshape(1, HID))

  return (h_gnn, h_node, h_sub, h_graph)
